# baseline (speedup 1.0000x reference)
import functools

import jax
import jax.numpy as jnp
from jax import lax
from jax.experimental import pallas as pl
from jax.experimental.pallas import tpu as pltpu
from jax.experimental.pallas import tpu_sc as plsc

LANES = 16
GROUP = 128
NC, NS = 2, 16
NW = NC * NS


def _tables_body(seg_ref, pos_ref, pos2_ref, delta_ref):
    L = pos_ref.shape[0]
    pos2_ref[0:L, :] = pos_ref[...] + seg_ref[0:1, :]
    pos2_ref[L:2 * L, :] = pos_ref[...] + seg_ref[0:1, :]
    delta_ref[...] = jnp.broadcast_to(seg_ref[1:2, :] - seg_ref[0:1, :],
                                      delta_ref.shape)


def _build_tables(segment_table, position_table):
    S, H = segment_table.shape
    L = position_table.shape[0]
    return pl.pallas_call(
        _tables_body,
        out_shape=[
            jax.ShapeDtypeStruct((2 * L, H), jnp.float32),
            jax.ShapeDtypeStruct((8, H), jnp.float32),
        ],
    )(segment_table, position_table)


def _sc_embed(xf, segf, token_table, pos2, delta, L, H, gpw):
    N = NW * gpw * GROUP
    mesh = plsc.VectorSubcoreMesh(core_axis_name="c", subcore_axis_name="s")

    @functools.partial(
        pl.kernel,
        mesh=mesh,
        out_type=jax.ShapeDtypeStruct((N, H), jnp.float32),
        scratch_types=[
            pltpu.VMEM((gpw, GROUP), jnp.int32),
            pltpu.VMEM((gpw, GROUP), jnp.int32),
            pltpu.VMEM((2 * L, H), jnp.float32),
            pltpu.VMEM((8, H), jnp.float32),
            pltpu.VMEM((GROUP, H), jnp.float32),
            pltpu.VMEM((GROUP, H), jnp.float32),
            pltpu.SemaphoreType.DMA,
            pltpu.SemaphoreType.DMA,
        ],
    )
    def k(xf_hbm, segf_hbm, tok_hbm, pos2_hbm, delta_hbm, out_hbm,
          idx_v, seg_v, pos2_v, delta_v, buf0, buf1, sem0, sem1):
        bufs = (buf0, buf1)
        sems = (sem0, sem1)
        wid = lax.axis_index("s") * NC + lax.axis_index("c")
        gbase = wid * gpw
        pltpu.sync_copy(xf_hbm.at[wid], idx_v)
        pltpu.sync_copy(segf_hbm.at[wid], seg_v)
        pltpu.sync_copy(pos2_hbm, pos2_v)
        pltpu.sync_copy(delta_hbm, delta_v)

        dregs = [delta_v[0, pl.ds(j * LANES, LANES)] for j in range(H // LANES)]

        def prefill(b, g):
            buf = bufs[b]
            p0 = lax.rem((gbase + g) * GROUP, L)

            def strip(s, carry):
                f = lax.convert_element_type(seg_v[g, pl.ds(s * LANES, LANES)],
                                             jnp.float32)
                for kk in range(LANES):
                    fk = lax.broadcast_in_dim(
                        lax.squeeze(lax.slice(f, (kk,), (kk + 1,)), (0,)),
                        (LANES,), ())
                    r = s * LANES + kk
                    for j in range(H // LANES):
                        sl = pl.ds(j * LANES, LANES)
                        buf[r, sl] = pos2_v[p0 + r, sl] + fk * dregs[j]
                return carry

            lax.fori_loop(0, GROUP // LANES, strip, 0)

        def issue_t(b, g):
            pltpu.async_copy(tok_hbm.at[idx_v.at[g]], bufs[b], sems[b], add=True)

        def issue_s(b, g):
            pltpu.async_copy(bufs[b], out_hbm.at[pl.ds((gbase + g) * GROUP, GROUP)],
                             sems[b])

        def wait_gather(b):
            pltpu.make_async_copy(pos2_hbm.at[pl.ds(0, GROUP)], bufs[b],
                                  sems[b]).wait()

        def wait_store(b):
            pltpu.make_async_copy(bufs[b], out_hbm.at[pl.ds(0, GROUP)],
                                  sems[b]).wait()

        def slot(b, g, wait_s, wait_t):
            if wait_s:
                wait_store(b)
            prefill(b, g)
            issue_t(b, g)
            if wait_t:
                b1 = 1 - b
                wait_gather(b1)
                issue_s(b1, g - 1)

        slot(0, 0, False, False)
        slot(1, 1, False, True)

        def outer_body(o, carry):
            g = o * 2
            slot(0, g, True, True)
            slot(1, g + 1, True, True)
            return carry

        lax.fori_loop(1, gpw // 2, outer_body, 0)

        wait_gather(1)
        issue_s(1, gpw - 1)
        wait_store(0)
        wait_store(1)

    return k(xf, segf, token_table, pos2, delta)


def kernel(x, segment_ids, token_table, segment_table, position_table):
    B, L = x.shape
    V, H = token_table.shape
    N = B * L
    assert N % (NW * GROUP) == 0
    gpw = N // (NW * GROUP)
    assert gpw % 2 == 0

    pos2, delta = _build_tables(segment_table, position_table)
    xf = x.reshape(NW, gpw, GROUP)
    segf = segment_ids.reshape(NW, gpw, GROUP)
    out = _sc_embed(xf, segf, token_table, pos2, delta, L, H, gpw)
    return out.reshape(B, L, H)
